# Initial kernel scaffold; baseline (speedup 1.0000x reference)
import jax
import jax.numpy as jnp
from jax.experimental import pallas as pl


def kernel(x, edge_index, W1, b1, W2, b2):
    raise NotImplementedError("write your pallas kernel here")


# speedup vs baseline: 1.0193x; 1.0193x over previous
import functools
import jax
import jax.numpy as jnp
from jax.experimental import pallas as pl
from jax.experimental.pallas import tpu as pltpu

N = 10000
E = 320000
F_IN = 128
NHID = 256
NCLS = 40
K = 10
ALPHA = 0.1

_ROWS = 1000


def _mlp_body(x_ref, w1_ref, b1_ref, w2_ref, b2_ref, z_ref):
    h = jnp.maximum(
        jnp.dot(x_ref[...], w1_ref[...], preferred_element_type=jnp.float32)
        + b1_ref[...],
        0.0,
    )
    z_ref[...] = (
        jnp.dot(h, w2_ref[...], preferred_element_type=jnp.float32) + b2_ref[...]
    )


def _mlp(x, W1, b1, W2, b2):
    grid = (N // _ROWS,)
    return pl.pallas_call(
        _mlp_body,
        grid=grid,
        in_specs=[
            pl.BlockSpec((_ROWS, F_IN), lambda i: (i, 0)),
            pl.BlockSpec((F_IN, NHID), lambda i: (0, 0)),
            pl.BlockSpec((1, NHID), lambda i: (0, 0)),
            pl.BlockSpec((NHID, NCLS), lambda i: (0, 0)),
            pl.BlockSpec((1, NCLS), lambda i: (0, 0)),
        ],
        out_specs=pl.BlockSpec((_ROWS, NCLS), lambda i: (i, 0)),
        out_shape=jax.ShapeDtypeStruct((N, NCLS), jnp.float32),
    )(x, W1, b1.reshape(1, NHID), W2, b2.reshape(1, NCLS))


def _lsm_body(v_ref, o_ref):
    v = v_ref[...]
    m = jnp.max(v, axis=1, keepdims=True)
    s = jnp.log(jnp.sum(jnp.exp(v - m), axis=1, keepdims=True))
    o_ref[...] = v - m - s


def _log_softmax(v):
    grid = (N // _ROWS,)
    return pl.pallas_call(
        _lsm_body,
        grid=grid,
        in_specs=[pl.BlockSpec((_ROWS, NCLS), lambda i: (i, 0))],
        out_specs=pl.BlockSpec((_ROWS, NCLS), lambda i: (i, 0)),
        out_shape=jax.ShapeDtypeStruct((N, NCLS), jnp.float32),
    )(v)


def kernel(x, edge_index, W1, b1, W2, b2):
    z = _mlp(x, W1, b1, W2, b2)

    src = edge_index[0]
    dst = edge_index[1]
    loop = jnp.arange(N, dtype=src.dtype)
    src = jnp.concatenate([src, loop])
    dst = jnp.concatenate([dst, loop])
    ones = jnp.ones(src.shape[0], dtype=z.dtype)
    deg = jax.ops.segment_sum(ones, dst, num_segments=N)
    dinv = jnp.where(deg > 0, deg ** -0.5, 0.0)
    norm = dinv[src] * dinv[dst]
    h0 = z
    xk = z
    for _ in range(K):
        msgs = xk[src] * norm[:, None]
        agg = jax.ops.segment_sum(msgs, dst, num_segments=N)
        xk = (1.0 - ALPHA) * agg + ALPHA * h0

    return _log_softmax(xk)


# speedup vs baseline: 9.1520x; 8.9784x over previous
import functools
import jax
import jax.numpy as jnp
from jax import lax
from jax.experimental import pallas as pl
from jax.experimental.pallas import tpu as pltpu
from jax.experimental.pallas import tpu_sc as plsc

N = 10000
E = 320000
F_IN = 128
NHID = 256
NCLS = 40
K = 10
ALPHA = 0.1

NS = 16
D = 48
N_PAD = 10240
RPT = N_PAD // NS
EC = 128
CPT = 157
EPT = EC * CPT
E_PAD = NS * EPT
TCROWS = 1024
LANES = 16

_mesh = plsc.VectorSubcoreMesh(core_axis_name="c", subcore_axis_name="s")
_sc_params = pltpu.CompilerParams(use_tc_tiling_on_sc=False)


def _mlp_body(x_ref, w1_ref, b1_ref, w2_ref, b2_ref, z_ref):
    h = jnp.maximum(
        jnp.dot(x_ref[...], w1_ref[...], preferred_element_type=jnp.float32)
        + b1_ref[...],
        0.0,
    )
    z = jnp.dot(h, w2_ref[...], preferred_element_type=jnp.float32) + b2_ref[...]
    z_ref[...] = jnp.pad(z, ((0, 0), (0, D - NCLS)))


def _mlp(xp, W1, b1, W2, b2):
    return pl.pallas_call(
        _mlp_body,
        grid=(N_PAD // TCROWS,),
        in_specs=[
            pl.BlockSpec((TCROWS, F_IN), lambda i: (i, 0)),
            pl.BlockSpec((F_IN, NHID), lambda i: (0, 0)),
            pl.BlockSpec((1, NHID), lambda i: (0, 0)),
            pl.BlockSpec((NHID, NCLS), lambda i: (0, 0)),
            pl.BlockSpec((1, NCLS), lambda i: (0, 0)),
        ],
        out_specs=pl.BlockSpec((TCROWS, D), lambda i: (i, 0)),
        out_shape=jax.ShapeDtypeStruct((N_PAD, D), jnp.float32),
    )(xp, W1, b1.reshape(1, NHID), W2, b2.reshape(1, NCLS))


@functools.partial(
    pl.kernel,
    mesh=_mesh,
    compiler_params=_sc_params,
    out_type=jax.ShapeDtypeStruct((N_PAD,), jnp.float32),
    scratch_types=[
        pltpu.VMEM((EC,), jnp.int32),
        pltpu.VMEM((EC,), jnp.float32),
        pltpu.VMEM((RPT,), jnp.float32),
        pltpu.VMEM_SHARED((N_PAD,), jnp.float32),
    ],
)
def _deg_kernel(dst_hbm, deg_hbm, didx_t, ones_t, zrow_t, deg_s):
    cid = lax.axis_index("c")
    sid = lax.axis_index("s")

    @pl.when(cid == 0)
    def _():
        for i in range(EC // LANES):
            ones_t[pl.ds(i * LANES, LANES)] = jnp.full((LANES,), 1.0, jnp.float32)
        for i in range(RPT // LANES):
            zrow_t[pl.ds(i * LANES, LANES)] = jnp.zeros((LANES,), jnp.float32)
        pltpu.sync_copy(zrow_t, deg_s.at[pl.ds(sid * RPT, RPT)])
        plsc.subcore_barrier()

        ebase = sid * EPT

        def chunk(ci, _):
            pltpu.sync_copy(dst_hbm.at[pl.ds(ebase + ci * EC, EC)], didx_t)
            pltpu.sync_copy(ones_t, deg_s.at[didx_t], add=True)
            return _

        lax.fori_loop(0, CPT, chunk, None)
        plsc.subcore_barrier()
        pltpu.sync_copy(
            deg_s.at[pl.ds(sid * RPT, RPT)], deg_hbm.at[pl.ds(sid * RPT, RPT)]
        )


def _prep_body(z_ref, deg_ref, u0_ref, cexp_ref):
    d = deg_ref[...] + 1.0
    dinv = jax.lax.rsqrt(d)
    u0_ref[...] = z_ref[...] * dinv
    cexp_ref[...] = jnp.broadcast_to((1.0 - ALPHA) / d, (TCROWS, D))


def _prep(zp, deg):
    return pl.pallas_call(
        _prep_body,
        grid=(N_PAD // TCROWS,),
        in_specs=[
            pl.BlockSpec((TCROWS, D), lambda i: (i, 0)),
            pl.BlockSpec((TCROWS, 1), lambda i: (i, 0)),
        ],
        out_specs=[
            pl.BlockSpec((TCROWS, D), lambda i: (i, 0)),
            pl.BlockSpec((TCROWS, D), lambda i: (i, 0)),
        ],
        out_shape=[
            jax.ShapeDtypeStruct((N_PAD, D), jnp.float32),
            jax.ShapeDtypeStruct((N_PAD, D), jnp.float32),
        ],
    )(zp, deg.reshape(N_PAD, 1))


@functools.partial(
    pl.kernel,
    mesh=_mesh,
    compiler_params=_sc_params,
    out_type=jax.ShapeDtypeStruct((N_PAD, D), jnp.float32),
    scratch_types=[
        pltpu.VMEM((RPT, D), jnp.float32),
        pltpu.VMEM((RPT, D), jnp.float32),
        pltpu.VMEM((EC,), jnp.int32),
        pltpu.VMEM((EC,), jnp.int32),
        pltpu.VMEM((EC, D), jnp.float32),
        pltpu.VMEM_SHARED((N_PAD, D), jnp.float32),
        pltpu.VMEM_SHARED((N_PAD, D), jnp.float32),
    ],
)
def _prop_kernel(
    u0_hbm,
    cexp_hbm,
    src_hbm,
    dst_hbm,
    out_hbm,
    g_t,
    cexp_t,
    sidx_t,
    didx_t,
    msg_t,
    u_s,
    acc_s,
):
    cid = lax.axis_index("c")
    sid = lax.axis_index("s")

    @pl.when(cid == 0)
    def _():
        r0 = sid * RPT
        rows = pl.ds(r0, RPT)
        ebase = sid * EPT

        pltpu.sync_copy(cexp_hbm.at[rows], cexp_t)
        pltpu.sync_copy(u0_hbm.at[rows], g_t)
        pltpu.sync_copy(g_t, u_s.at[rows])
        pltpu.sync_copy(g_t, acc_s.at[rows])

        def gscale(r, _g):
            for c in range(D // LANES):
                sl = pl.ds(c * LANES, LANES)
                g_t[r, sl] = g_t[r, sl] * ALPHA
            return _g

        lax.fori_loop(0, RPT, gscale, None)
        plsc.subcore_barrier()

        def step(_k, _):
            def chunk(ci, _c):
                base = ebase + ci * EC
                pltpu.sync_copy(src_hbm.at[pl.ds(base, EC)], sidx_t)
                pltpu.sync_copy(dst_hbm.at[pl.ds(base, EC)], didx_t)
                pltpu.sync_copy(u_s.at[sidx_t], msg_t)
                pltpu.sync_copy(msg_t, acc_s.at[didx_t], add=True)
                return _c

            lax.fori_loop(0, CPT, chunk, None)
            plsc.subcore_barrier()

            def ew_chunk(j, _e):
                rr = pl.ds(r0 + j * EC, EC)

                def ewrow(r, _2):
                    for c in range(D // LANES):
                        sl = pl.ds(c * LANES, LANES)
                        msg_t[r, sl] = (
                            cexp_t[j * EC + r, sl] * msg_t[r, sl]
                            + g_t[j * EC + r, sl]
                        )
                    return _2

                pltpu.sync_copy(acc_s.at[rr], msg_t)
                lax.fori_loop(0, EC, ewrow, None)
                pltpu.sync_copy(msg_t, u_s.at[rr])
                pltpu.sync_copy(msg_t, acc_s.at[rr])
                return _e

            lax.fori_loop(0, RPT // EC, ew_chunk, None)
            plsc.subcore_barrier()
            return _

        lax.fori_loop(0, K, step, None)

        pltpu.sync_copy(u_s.at[rows], out_hbm.at[rows])


def _final_body(u_ref, deg_ref, o_ref):
    v = u_ref[...][:, :NCLS] * jnp.sqrt(deg_ref[...] + 1.0)
    m = jnp.max(v, axis=1, keepdims=True)
    s = jnp.log(jnp.sum(jnp.exp(v - m), axis=1, keepdims=True))
    o_ref[...] = v - m - s


def _final(uK, deg):
    rows = 1000
    return pl.pallas_call(
        _final_body,
        grid=(N // rows,),
        in_specs=[
            pl.BlockSpec((rows, D), lambda i: (i, 0)),
            pl.BlockSpec((rows, 1), lambda i: (i, 0)),
        ],
        out_specs=pl.BlockSpec((rows, NCLS), lambda i: (i, 0)),
        out_shape=jax.ShapeDtypeStruct((N, NCLS), jnp.float32),
    )(uK, deg.reshape(N_PAD, 1))


def kernel(x, edge_index, W1, b1, W2, b2):
    xp = jnp.pad(x, ((0, N_PAD - N), (0, 0)))
    pad_e = E_PAD - E
    pad_idx = N + (jnp.arange(pad_e, dtype=jnp.int32) % (N_PAD - N))
    src = jnp.concatenate([edge_index[0], pad_idx])
    dst = jnp.concatenate([edge_index[1], pad_idx])

    zp = _mlp(xp, W1, b1, W2, b2)
    deg = _deg_kernel(dst)
    u0, cexp = _prep(zp, deg)
    uK = _prop_kernel(u0, cexp, src, dst)
    return _final(uK, deg)


# speedup vs baseline: 20.4918x; 2.2390x over previous
import functools
import jax
import jax.numpy as jnp
from jax import lax
from jax.experimental import pallas as pl
from jax.experimental.pallas import tpu as pltpu
from jax.experimental.pallas import tpu_sc as plsc

N = 10000
E = 320000
F_IN = 128
NHID = 256
NCLS = 40
K = 10
ALPHA = 0.1

NS = 16
D = 48
N_PAD = 10240
RPT = N_PAD // NS
EC = 128
CPT = 160
EPT = EC * CPT
E_PAD = NS * EPT
TCROWS = 1024
LANES = 16

_mesh = plsc.VectorSubcoreMesh(core_axis_name="c", subcore_axis_name="s")
_sc_params = pltpu.CompilerParams(
    use_tc_tiling_on_sc=False, needs_layout_passes=False
)


def _mlp_body(x_ref, w1_ref, b1_ref, w2_ref, b2_ref, z_ref):
    h = jnp.maximum(
        jnp.dot(x_ref[...], w1_ref[...], preferred_element_type=jnp.float32)
        + b1_ref[...],
        0.0,
    )
    z = jnp.dot(h, w2_ref[...], preferred_element_type=jnp.float32) + b2_ref[...]
    z_ref[...] = jnp.pad(z, ((0, 0), (0, D - NCLS)))


def _mlp(xp, W1, b1, W2, b2):
    return pl.pallas_call(
        _mlp_body,
        grid=(N_PAD // TCROWS,),
        in_specs=[
            pl.BlockSpec((TCROWS, F_IN), lambda i: (i, 0)),
            pl.BlockSpec((F_IN, NHID), lambda i: (0, 0)),
            pl.BlockSpec((1, NHID), lambda i: (0, 0)),
            pl.BlockSpec((NHID, NCLS), lambda i: (0, 0)),
            pl.BlockSpec((1, NCLS), lambda i: (0, 0)),
        ],
        out_specs=pl.BlockSpec((TCROWS, D), lambda i: (i, 0)),
        out_shape=jax.ShapeDtypeStruct((N_PAD, D), jnp.float32),
    )(xp, W1, b1.reshape(1, NHID), W2, b2.reshape(1, NCLS))


@functools.partial(
    pl.kernel,
    mesh=_mesh,
    compiler_params=_sc_params,
    out_type=jax.ShapeDtypeStruct((N_PAD,), jnp.float32),
    scratch_types=[
        pltpu.VMEM((EC,), jnp.int32),
        pltpu.VMEM((EC,), jnp.float32),
        pltpu.VMEM((RPT,), jnp.float32),
        pltpu.VMEM_SHARED((N_PAD,), jnp.float32),
    ],
)
def _deg_kernel(dst_hbm, deg_hbm, didx_t, ones_t, zrow_t, deg_s):
    cid = lax.axis_index("c")
    sid = lax.axis_index("s")

    @pl.when(cid == 0)
    def _():
        for i in range(EC // LANES):
            ones_t[pl.ds(i * LANES, LANES)] = jnp.full((LANES,), 1.0, jnp.float32)
        for i in range(RPT // LANES):
            zrow_t[pl.ds(i * LANES, LANES)] = jnp.zeros((LANES,), jnp.float32)
        pltpu.sync_copy(zrow_t, deg_s.at[pl.ds(sid * RPT, RPT)])
        plsc.subcore_barrier()

        def chunk(ci, _):
            pltpu.sync_copy(dst_hbm.at[sid * CPT + ci], didx_t)
            pltpu.sync_copy(ones_t, deg_s.at[didx_t], add=True)
            return _

        lax.fori_loop(0, CPT, chunk, None)
        plsc.subcore_barrier()
        pltpu.sync_copy(
            deg_s.at[pl.ds(sid * RPT, RPT)], deg_hbm.at[pl.ds(sid * RPT, RPT)]
        )


def _prep_body(z_ref, deg_ref, u0_ref):
    d = deg_ref[...] + 1.0
    u0_ref[...] = z_ref[...] * jax.lax.rsqrt(d)


def _prep(zp, deg):
    return pl.pallas_call(
        _prep_body,
        grid=(N_PAD // TCROWS,),
        in_specs=[
            pl.BlockSpec((TCROWS, D), lambda i: (i, 0)),
            pl.BlockSpec((TCROWS, 1), lambda i: (i, 0)),
        ],
        out_specs=pl.BlockSpec((TCROWS, D), lambda i: (i, 0)),
        out_shape=jax.ShapeDtypeStruct((N_PAD, D), jnp.float32),
    )(zp, deg.reshape(N_PAD, 1))


@functools.partial(
    pl.kernel,
    mesh=_mesh,
    compiler_params=_sc_params,
    out_type=jax.ShapeDtypeStruct((N_PAD, D), jnp.float32),
    scratch_types=[
        pltpu.VMEM((RPT,), jnp.float32),
        pltpu.VMEM((RPT,), jnp.float32),
        pltpu.VMEM((CPT, EC), jnp.int32),
        pltpu.VMEM((CPT, EC), jnp.int32),
        pltpu.VMEM((EC, D), jnp.float32),
        pltpu.VMEM((EC, D), jnp.float32),
        pltpu.VMEM((EC, D), jnp.float32),
        pltpu.SemaphoreType.DMA,
        pltpu.SemaphoreType.DMA,
        pltpu.SemaphoreType.DMA,
        pltpu.SemaphoreType.DMA,
        pltpu.VMEM_SHARED((N_PAD, D), jnp.float32),
        pltpu.VMEM_SHARED((N_PAD, D), jnp.float32),
    ],
)
def _prop_kernel(
    u0_hbm,
    deg_hbm,
    src_hbm,
    dst_hbm,
    out_hbm,
    cexp_r,
    degb,
    sidx_t,
    didx_t,
    msg0,
    msg1,
    gst,
    gsem0,
    gsem1,
    ssem0,
    ssem1,
    u_s,
    acc_s,
):
    cid = lax.axis_index("c")
    sid = lax.axis_index("s")
    msgs = (msg0, msg1)
    gsems = (gsem0, gsem1)
    ssems = (ssem0, ssem1)

    @pl.when(cid == 0)
    def _():
        r0 = sid * RPT
        rows = pl.ds(r0, RPT)

        pltpu.sync_copy(src_hbm.at[pl.ds(sid * CPT, CPT)], sidx_t)
        pltpu.sync_copy(dst_hbm.at[pl.ds(sid * CPT, CPT)], didx_t)
        pltpu.sync_copy(deg_hbm.at[rows], degb)

        def cinit(i, _c):
            sl = pl.ds(i * LANES, LANES)
            cexp_r[sl] = (1.0 - ALPHA) / (degb[sl] + 1.0)
            return _c

        lax.fori_loop(0, RPT // LANES, cinit, None)

        def seed(j, _s):
            rr = pl.ds(r0 + j * EC, EC)
            pltpu.sync_copy(u0_hbm.at[rr], gst)
            pltpu.sync_copy(gst, u_s.at[rr])
            pltpu.sync_copy(gst, acc_s.at[rr])
            return _s

        lax.fori_loop(0, RPT // EC, seed, None)
        plsc.subcore_barrier()

        def gather_start(c, p):
            pltpu.async_copy(u_s.at[sidx_t.at[c]], msgs[p], gsems[p])

        def gather_wait(c, p):
            pltpu.make_async_copy(u_s.at[sidx_t.at[c]], msgs[p], gsems[p]).wait()

        def scat_start(c, p):
            pltpu.async_copy(msgs[p], acc_s.at[didx_t.at[c]], ssems[p], add=True)

        def scat_wait(c, p):
            pltpu.make_async_copy(
                msgs[p], acc_s.at[didx_t.at[c]], ssems[p]
            ).wait()

        def step(_k, _):
            gather_start(0, 0)
            gather_start(1, 1)

            def blk(b, _c):
                for p in range(2):
                    c = 2 * b + p
                    gather_wait(c, p)
                    scat_start(c, p)
                    scat_wait(c, p)

                    @pl.when(c + 2 < CPT)
                    def _pref():
                        gather_start(c + 2, p)

                return _c

            lax.fori_loop(0, CPT // 2, blk, None)
            plsc.subcore_barrier()

            def ew_chunk(j, _e):
                rr = pl.ds(r0 + j * EC, EC)
                pltpu.sync_copy(acc_s.at[rr], msg0)
                pltpu.sync_copy(u0_hbm.at[rr], gst)

                def ewrow(r, _2):
                    bc = plsc.load_gather(
                        cexp_r, [jnp.full((LANES,), j * EC + r, jnp.int32)]
                    )
                    for c in range(D // LANES):
                        sl = pl.ds(c * LANES, LANES)
                        msg0[r, sl] = bc * msg0[r, sl] + ALPHA * gst[r, sl]
                    return _2

                lax.fori_loop(0, EC, ewrow, None)
                pltpu.sync_copy(msg0, u_s.at[rr])
                pltpu.sync_copy(msg0, acc_s.at[rr])
                return _e

            lax.fori_loop(0, RPT // EC, ew_chunk, None)
            plsc.subcore_barrier()
            return _

        lax.fori_loop(0, K, step, None)

        pltpu.sync_copy(u_s.at[rows], out_hbm.at[rows])


def _final_body(u_ref, deg_ref, o_ref):
    v = u_ref[...][:, :NCLS] * jnp.sqrt(deg_ref[...] + 1.0)
    m = jnp.max(v, axis=1, keepdims=True)
    s = jnp.log(jnp.sum(jnp.exp(v - m), axis=1, keepdims=True))
    o_ref[...] = v - m - s


def _final(uK, deg):
    rows = 1000
    return pl.pallas_call(
        _final_body,
        grid=(N // rows,),
        in_specs=[
            pl.BlockSpec((rows, D), lambda i: (i, 0)),
            pl.BlockSpec((rows, 1), lambda i: (i, 0)),
        ],
        out_specs=pl.BlockSpec((rows, NCLS), lambda i: (i, 0)),
        out_shape=jax.ShapeDtypeStruct((N, NCLS), jnp.float32),
    )(uK, deg.reshape(N_PAD, 1))


def kernel(x, edge_index, W1, b1, W2, b2):
    xp = jnp.pad(x, ((0, N_PAD - N), (0, 0)))
    pad_e = E_PAD - E
    pad_idx = N + (jnp.arange(pad_e, dtype=jnp.int32) % (N_PAD - N))
    src = jnp.concatenate([edge_index[0], pad_idx]).reshape(NS * CPT, EC)
    dst = jnp.concatenate([edge_index[1], pad_idx]).reshape(NS * CPT, EC)

    zp = _mlp(xp, W1, b1, W2, b2)
    deg = _deg_kernel(dst)
    u0 = _prep(zp, deg)
    uK = _prop_kernel(u0, deg, src, dst)
    return _final(uK, deg)


# speedup vs baseline: 24.1012x; 1.1761x over previous
import functools
import jax
import jax.numpy as jnp
from jax import lax
from jax.experimental import pallas as pl
from jax.experimental.pallas import tpu as pltpu
from jax.experimental.pallas import tpu_sc as plsc

N = 10000
E = 320000
F_IN = 128
NHID = 256
NCLS = 40
K = 10
ALPHA = 0.1

NS = 16
D = 48
N_PAD = 10240
RPT = N_PAD // NS
EC = 128
CPT = 160
EPT = EC * CPT
E_PAD = NS * EPT
TCROWS = 1024
LANES = 16

_mesh = plsc.VectorSubcoreMesh(core_axis_name="c", subcore_axis_name="s")
_sc_params = pltpu.CompilerParams(
    use_tc_tiling_on_sc=False, needs_layout_passes=False
)


def _mlp_body(x_ref, w1_ref, b1_ref, w2_ref, b2_ref, z_ref):
    h = jnp.maximum(
        jnp.dot(x_ref[...], w1_ref[...], preferred_element_type=jnp.float32)
        + b1_ref[...],
        0.0,
    )
    z = jnp.dot(h, w2_ref[...], preferred_element_type=jnp.float32) + b2_ref[...]
    z_ref[...] = jnp.pad(z, ((0, 0), (0, D - NCLS)))


def _mlp(xp, W1, b1, W2, b2):
    return pl.pallas_call(
        _mlp_body,
        grid=(N_PAD // TCROWS,),
        in_specs=[
            pl.BlockSpec((TCROWS, F_IN), lambda i: (i, 0)),
            pl.BlockSpec((F_IN, NHID), lambda i: (0, 0)),
            pl.BlockSpec((1, NHID), lambda i: (0, 0)),
            pl.BlockSpec((NHID, NCLS), lambda i: (0, 0)),
            pl.BlockSpec((1, NCLS), lambda i: (0, 0)),
        ],
        out_specs=pl.BlockSpec((TCROWS, D), lambda i: (i, 0)),
        out_shape=jax.ShapeDtypeStruct((N_PAD, D), jnp.float32),
    )(xp, W1, b1.reshape(1, NHID), W2, b2.reshape(1, NCLS))


@functools.partial(
    pl.kernel,
    mesh=_mesh,
    compiler_params=_sc_params,
    out_type=jax.ShapeDtypeStruct((N_PAD,), jnp.float32),
    scratch_types=[
        pltpu.VMEM((EC,), jnp.int32),
        pltpu.VMEM((EC,), jnp.float32),
        pltpu.VMEM((RPT,), jnp.float32),
        pltpu.VMEM_SHARED((N_PAD,), jnp.float32),
    ],
)
def _deg_kernel(dst_hbm, deg_hbm, didx_t, ones_t, zrow_t, deg_s):
    cid = lax.axis_index("c")
    sid = lax.axis_index("s")

    @pl.when(cid == 0)
    def _():
        for i in range(EC // LANES):
            ones_t[pl.ds(i * LANES, LANES)] = jnp.full((LANES,), 1.0, jnp.float32)
        for i in range(RPT // LANES):
            zrow_t[pl.ds(i * LANES, LANES)] = jnp.zeros((LANES,), jnp.float32)
        pltpu.sync_copy(zrow_t, deg_s.at[pl.ds(sid * RPT, RPT)])
        plsc.subcore_barrier()

        def chunk(ci, _):
            pltpu.sync_copy(dst_hbm.at[sid * CPT + ci], didx_t)
            pltpu.sync_copy(ones_t, deg_s.at[didx_t], add=True)
            return _

        lax.fori_loop(0, CPT, chunk, None)
        plsc.subcore_barrier()
        pltpu.sync_copy(
            deg_s.at[pl.ds(sid * RPT, RPT)], deg_hbm.at[pl.ds(sid * RPT, RPT)]
        )


def _prep_body(z_ref, deg_ref, u0_ref):
    d = deg_ref[...] + 1.0
    u0_ref[...] = z_ref[...] * jax.lax.rsqrt(d)


def _prep(zp, deg):
    return pl.pallas_call(
        _prep_body,
        grid=(N_PAD // TCROWS,),
        in_specs=[
            pl.BlockSpec((TCROWS, D), lambda i: (i, 0)),
            pl.BlockSpec((TCROWS, 1), lambda i: (i, 0)),
        ],
        out_specs=pl.BlockSpec((TCROWS, D), lambda i: (i, 0)),
        out_shape=jax.ShapeDtypeStruct((N_PAD, D), jnp.float32),
    )(zp, deg.reshape(N_PAD, 1))


@functools.partial(
    pl.kernel,
    mesh=_mesh,
    compiler_params=_sc_params,
    out_type=jax.ShapeDtypeStruct((N_PAD, D), jnp.float32),
    scratch_types=[
        pltpu.VMEM((RPT,), jnp.float32),
        pltpu.VMEM((RPT,), jnp.float32),
        pltpu.VMEM((CPT, EC), jnp.int32),
        pltpu.VMEM((CPT, EC), jnp.int32),
        pltpu.VMEM((EC, D), jnp.float32),
        pltpu.VMEM((EC, D), jnp.float32),
        pltpu.VMEM((EC, D), jnp.float32),
        pltpu.VMEM((EC, D), jnp.float32),
        pltpu.SemaphoreType.DMA,
        pltpu.SemaphoreType.DMA,
        pltpu.SemaphoreType.DMA,
        pltpu.SemaphoreType.DMA,
        pltpu.SemaphoreType.DMA,
        pltpu.SemaphoreType.DMA,
        pltpu.SemaphoreType.DMA,
        pltpu.SemaphoreType.DMA,
        pltpu.VMEM_SHARED((N_PAD, D), jnp.float32),
        pltpu.VMEM_SHARED((N_PAD, D), jnp.float32),
    ],
)
def _prop_kernel(
    u0_hbm,
    deg_hbm,
    src_hbm,
    dst_hbm,
    out_hbm,
    cexp_r,
    degb,
    sidx_t,
    didx_t,
    msg0,
    msg1,
    msg2,
    msg3,
    gsem0,
    gsem1,
    gsem2,
    gsem3,
    ssem0,
    ssem1,
    ssem2,
    ssem3,
    u_s,
    acc_s,
):
    cid = lax.axis_index("c")
    sid = lax.axis_index("s")
    msgs = (msg0, msg1, msg2, msg3)
    gsems = (gsem0, gsem1, gsem2, gsem3)
    ssems = (ssem0, ssem1, ssem2, ssem3)

    @pl.when(cid == 0)
    def _():
        r0 = sid * RPT
        rows = pl.ds(r0, RPT)

        pltpu.sync_copy(src_hbm.at[pl.ds(sid * CPT, CPT)], sidx_t)
        pltpu.sync_copy(dst_hbm.at[pl.ds(sid * CPT, CPT)], didx_t)
        pltpu.sync_copy(deg_hbm.at[rows], degb)

        def cinit(i, _c):
            sl = pl.ds(i * LANES, LANES)
            cexp_r[sl] = (1.0 - ALPHA) / (degb[sl] + 1.0)
            return _c

        lax.fori_loop(0, RPT // LANES, cinit, None)

        def seed(j, _s):
            rr = pl.ds(r0 + j * EC, EC)
            pltpu.sync_copy(u0_hbm.at[rr], msg0)
            pltpu.sync_copy(msg0, u_s.at[rr])
            pltpu.sync_copy(msg0, acc_s.at[rr])
            return _s

        lax.fori_loop(0, RPT // EC, seed, None)
        plsc.subcore_barrier()

        def gather_start(c, p):
            pltpu.async_copy(u_s.at[sidx_t.at[c]], msgs[p], gsems[p])

        def gather_wait(c, p):
            pltpu.make_async_copy(u_s.at[sidx_t.at[c]], msgs[p], gsems[p]).wait()

        def scat_start(c, p):
            pltpu.async_copy(msgs[p], acc_s.at[didx_t.at[c]], ssems[p], add=True)

        def scat_wait(c, p):
            pltpu.make_async_copy(
                msgs[p], acc_s.at[didx_t.at[c]], ssems[p]
            ).wait()

        NEW = RPT // EC

        def ew_in_start(j, q):
            rr = pl.ds(r0 + j * EC, EC)
            pltpu.async_copy(acc_s.at[rr], msgs[2 * q], gsems[2 * q])
            pltpu.async_copy(u0_hbm.at[rr], msgs[2 * q + 1], gsems[2 * q + 1])

        def ew_in_wait(j, q):
            rr = pl.ds(r0 + j * EC, EC)
            pltpu.make_async_copy(acc_s.at[rr], msgs[2 * q], gsems[2 * q]).wait()
            pltpu.make_async_copy(
                u0_hbm.at[rr], msgs[2 * q + 1], gsems[2 * q + 1]
            ).wait()

        def ew_out_start(j, q):
            rr = pl.ds(r0 + j * EC, EC)
            pltpu.async_copy(msgs[2 * q], u_s.at[rr], ssems[2 * q])
            pltpu.async_copy(msgs[2 * q], acc_s.at[rr], ssems[2 * q + 1])

        def ew_out_wait(j, q):
            rr = pl.ds(r0 + j * EC, EC)
            pltpu.make_async_copy(msgs[2 * q], u_s.at[rr], ssems[2 * q]).wait()
            pltpu.make_async_copy(
                msgs[2 * q], acc_s.at[rr], ssems[2 * q + 1]
            ).wait()

        def step(_k, _):
            for p in range(4):
                gather_start(p, p)

            def blk(b, _c):
                for p in range(4):
                    c = 4 * b + p
                    gather_wait(c, p)
                    scat_start(c, p)

                    @pl.when(jnp.logical_and(c >= 2, c + 2 < CPT))
                    def _pref(c=c, p=p):
                        scat_wait(c - 2, (p - 2) % 4)
                        gather_start(c + 2, (p + 2) % 4)

                return _c

            lax.fori_loop(0, CPT // 4, blk, None)
            for p in range(4):
                scat_wait(CPT - 4 + p, p)
            plsc.subcore_barrier()

            ew_in_start(0, 0)
            ew_in_start(1, 1)

            for jj in range(NEW):
                q = jj % 2
                ew_in_wait(jj, q)

                def ewrow(r, _2, jj=jj, q=q):
                    bc = plsc.load_gather(
                        cexp_r,
                        [jnp.full((LANES,), jj * EC + r, jnp.int32)],
                    )
                    for c in range(D // LANES):
                        sl = pl.ds(c * LANES, LANES)
                        msgs[2 * q][r, sl] = (
                            bc * msgs[2 * q][r, sl]
                            + ALPHA * msgs[2 * q + 1][r, sl]
                        )
                    return _2

                lax.fori_loop(0, EC, ewrow, None, unroll=2)
                ew_out_start(jj, q)
                if jj >= 1:
                    ew_out_wait(jj - 1, 1 - q)
                    if jj + 1 < NEW:
                        ew_in_start(jj + 1, 1 - q)
            ew_out_wait(NEW - 1, (NEW - 1) % 2)
            plsc.subcore_barrier()
            return _

        lax.fori_loop(0, K, step, None)

        pltpu.sync_copy(u_s.at[rows], out_hbm.at[rows])


def _final_body(u_ref, deg_ref, o_ref):
    v = u_ref[...][:, :NCLS] * jnp.sqrt(deg_ref[...] + 1.0)
    m = jnp.max(v, axis=1, keepdims=True)
    s = jnp.log(jnp.sum(jnp.exp(v - m), axis=1, keepdims=True))
    o_ref[...] = v - m - s


def _final(uK, deg):
    rows = 1000
    return pl.pallas_call(
        _final_body,
        grid=(N // rows,),
        in_specs=[
            pl.BlockSpec((rows, D), lambda i: (i, 0)),
            pl.BlockSpec((rows, 1), lambda i: (i, 0)),
        ],
        out_specs=pl.BlockSpec((rows, NCLS), lambda i: (i, 0)),
        out_shape=jax.ShapeDtypeStruct((N, NCLS), jnp.float32),
    )(uK, deg.reshape(N_PAD, 1))


def kernel(x, edge_index, W1, b1, W2, b2):
    xp = jnp.pad(x, ((0, N_PAD - N), (0, 0)))
    pad_e = E_PAD - E
    pad_idx = N + (jnp.arange(pad_e, dtype=jnp.int32) % (N_PAD - N))
    src = jnp.concatenate([edge_index[0], pad_idx]).reshape(NS * CPT, EC)
    dst = jnp.concatenate([edge_index[1], pad_idx]).reshape(NS * CPT, EC)

    zp = _mlp(xp, W1, b1, W2, b2)
    deg = _deg_kernel(dst)
    u0 = _prep(zp, deg)
    uK = _prop_kernel(u0, deg, src, dst)
    return _final(uK, deg)


# speedup vs baseline: 30.8874x; 1.2816x over previous
import functools
import jax
import jax.numpy as jnp
from jax import lax
from jax.experimental import pallas as pl
from jax.experimental.pallas import tpu as pltpu
from jax.experimental.pallas import tpu_sc as plsc

N = 10000
E = 320000
F_IN = 128
NHID = 256
NCLS = 40
K = 10
ALPHA = 0.1

NS = 16
D = NCLS
N_PAD = 10240
RPT = N_PAD // NS
EC = 128
CPT = 160
EPT = EC * CPT
E_PAD = NS * EPT
TCROWS = 1024
LANES = 16
_COLS = (0, 16, 24)

_mesh = plsc.VectorSubcoreMesh(core_axis_name="c", subcore_axis_name="s")
_sc_params = pltpu.CompilerParams(
    use_tc_tiling_on_sc=False, needs_layout_passes=False
)


def _mlp_body(x_ref, w1_ref, b1_ref, w2_ref, b2_ref, z_ref):
    h = jnp.maximum(
        jnp.dot(x_ref[...], w1_ref[...], preferred_element_type=jnp.float32)
        + b1_ref[...],
        0.0,
    )
    z_ref[...] = (
        jnp.dot(h, w2_ref[...], preferred_element_type=jnp.float32) + b2_ref[...]
    )


def _mlp(xp, W1, b1, W2, b2):
    return pl.pallas_call(
        _mlp_body,
        grid=(N_PAD // TCROWS,),
        in_specs=[
            pl.BlockSpec((TCROWS, F_IN), lambda i: (i, 0)),
            pl.BlockSpec((F_IN, NHID), lambda i: (0, 0)),
            pl.BlockSpec((1, NHID), lambda i: (0, 0)),
            pl.BlockSpec((NHID, NCLS), lambda i: (0, 0)),
            pl.BlockSpec((1, NCLS), lambda i: (0, 0)),
        ],
        out_specs=pl.BlockSpec((TCROWS, D), lambda i: (i, 0)),
        out_shape=jax.ShapeDtypeStruct((N_PAD, D), jnp.float32),
    )(xp, W1, b1.reshape(1, NHID), W2, b2.reshape(1, NCLS))


@functools.partial(
    pl.kernel,
    mesh=_mesh,
    compiler_params=_sc_params,
    out_type=jax.ShapeDtypeStruct((N_PAD,), jnp.float32),
    scratch_types=[
        pltpu.VMEM((4, EC), jnp.int32),
        pltpu.VMEM((EC,), jnp.float32),
        pltpu.VMEM((RPT,), jnp.float32),
        pltpu.SemaphoreType.DMA,
        pltpu.SemaphoreType.DMA,
        pltpu.SemaphoreType.DMA,
        pltpu.SemaphoreType.DMA,
        pltpu.SemaphoreType.DMA,
        pltpu.SemaphoreType.DMA,
        pltpu.SemaphoreType.DMA,
        pltpu.SemaphoreType.DMA,
        pltpu.VMEM_SHARED((N_PAD,), jnp.float32),
    ],
)
def _deg_kernel(
    dst_hbm,
    deg_hbm,
    didx_t,
    ones_t,
    zrow_t,
    l0,
    l1,
    l2,
    l3,
    s0,
    s1,
    s2,
    s3,
    deg_s,
):
    cid = lax.axis_index("c")
    sid = lax.axis_index("s")
    lsems = (l0, l1, l2, l3)
    ssems = (s0, s1, s2, s3)

    @pl.when(cid == 0)
    def _():
        for i in range(EC // LANES):
            ones_t[pl.ds(i * LANES, LANES)] = jnp.full((LANES,), 1.0, jnp.float32)
        for i in range(RPT // LANES):
            zrow_t[pl.ds(i * LANES, LANES)] = jnp.zeros((LANES,), jnp.float32)
        pltpu.sync_copy(zrow_t, deg_s.at[pl.ds(sid * RPT, RPT)])
        plsc.subcore_barrier()

        def lstart(c, p):
            pltpu.async_copy(dst_hbm.at[sid * CPT + c], didx_t.at[p], lsems[p])

        def lwait(c, p):
            pltpu.make_async_copy(
                dst_hbm.at[sid * CPT + c], didx_t.at[p], lsems[p]
            ).wait()

        def sstart(p):
            pltpu.async_copy(ones_t, deg_s.at[didx_t.at[p]], ssems[p], add=True)

        def swait(p):
            pltpu.make_async_copy(ones_t, deg_s.at[didx_t.at[p]], ssems[p]).wait()

        for p in range(4):
            lstart(p, p)

        def blk(b, _c):
            for p in range(4):
                c = 4 * b + p
                lwait(c, p)
                sstart(p)

                @pl.when(jnp.logical_and(c >= 2, c + 2 < CPT))
                def _pref(c=c, p=p):
                    swait((p - 2) % 4)
                    lstart(c + 2, (p + 2) % 4)

            return _c

        lax.fori_loop(0, CPT // 4, blk, None)
        for p in range(4):
            swait(p)
        plsc.subcore_barrier()
        pltpu.sync_copy(
            deg_s.at[pl.ds(sid * RPT, RPT)], deg_hbm.at[pl.ds(sid * RPT, RPT)]
        )


def _prep_body(z_ref, deg_ref, u0_ref):
    d = deg_ref[...] + 1.0
    u0_ref[...] = z_ref[...] * jax.lax.rsqrt(d)


def _prep(zp, deg):
    return pl.pallas_call(
        _prep_body,
        grid=(N_PAD // TCROWS,),
        in_specs=[
            pl.BlockSpec((TCROWS, D), lambda i: (i, 0)),
            pl.BlockSpec((TCROWS, 1), lambda i: (i, 0)),
        ],
        out_specs=pl.BlockSpec((TCROWS, D), lambda i: (i, 0)),
        out_shape=jax.ShapeDtypeStruct((N_PAD, D), jnp.float32),
    )(zp, deg.reshape(N_PAD, 1))


@functools.partial(
    pl.kernel,
    mesh=_mesh,
    compiler_params=_sc_params,
    out_type=jax.ShapeDtypeStruct((N_PAD, D), jnp.float32),
    scratch_types=[
        pltpu.VMEM((RPT,), jnp.float32),
        pltpu.VMEM((RPT,), jnp.float32),
        pltpu.VMEM((CPT, EC), jnp.int32),
        pltpu.VMEM((CPT, EC), jnp.int32),
        pltpu.VMEM((EC, D), jnp.float32),
        pltpu.VMEM((EC, D), jnp.float32),
        pltpu.VMEM((EC, D), jnp.float32),
        pltpu.VMEM((EC, D), jnp.float32),
        pltpu.SemaphoreType.DMA,
        pltpu.SemaphoreType.DMA,
        pltpu.SemaphoreType.DMA,
        pltpu.SemaphoreType.DMA,
        pltpu.SemaphoreType.DMA,
        pltpu.SemaphoreType.DMA,
        pltpu.SemaphoreType.DMA,
        pltpu.SemaphoreType.DMA,
        pltpu.VMEM_SHARED((N_PAD, D), jnp.float32),
        pltpu.VMEM_SHARED((N_PAD, D), jnp.float32),
    ],
)
def _prop_kernel(
    u0_hbm,
    deg_hbm,
    src_hbm,
    dst_hbm,
    out_hbm,
    cexp_r,
    degb,
    sidx_t,
    didx_t,
    msg0,
    msg1,
    msg2,
    msg3,
    gsem0,
    gsem1,
    gsem2,
    gsem3,
    ssem0,
    ssem1,
    ssem2,
    ssem3,
    u_s,
    acc_s,
):
    cid = lax.axis_index("c")
    sid = lax.axis_index("s")
    msgs = (msg0, msg1, msg2, msg3)
    gsems = (gsem0, gsem1, gsem2, gsem3)
    ssems = (ssem0, ssem1, ssem2, ssem3)

    @pl.when(cid == 0)
    def _():
        r0 = sid * RPT
        rows = pl.ds(r0, RPT)

        pltpu.sync_copy(src_hbm.at[pl.ds(sid * CPT, CPT)], sidx_t)
        pltpu.sync_copy(dst_hbm.at[pl.ds(sid * CPT, CPT)], didx_t)
        pltpu.sync_copy(deg_hbm.at[rows], degb)

        def cinit(i, _c):
            sl = pl.ds(i * LANES, LANES)
            cexp_r[sl] = (1.0 - ALPHA) / (degb[sl] + 1.0)
            return _c

        lax.fori_loop(0, RPT // LANES, cinit, None)

        def seed(j, _s):
            rr = pl.ds(r0 + j * EC, EC)
            pltpu.sync_copy(u0_hbm.at[rr], msg0)
            pltpu.sync_copy(msg0, u_s.at[rr])
            pltpu.sync_copy(msg0, acc_s.at[rr])
            return _s

        lax.fori_loop(0, RPT // EC, seed, None)
        plsc.subcore_barrier()

        def gather_start(c, p):
            pltpu.async_copy(u_s.at[sidx_t.at[c]], msgs[p], gsems[p])

        def gather_wait(c, p):
            pltpu.make_async_copy(u_s.at[sidx_t.at[c]], msgs[p], gsems[p]).wait()

        def scat_start(c, p):
            pltpu.async_copy(msgs[p], acc_s.at[didx_t.at[c]], ssems[p], add=True)

        def scat_wait(c, p):
            pltpu.make_async_copy(
                msgs[p], acc_s.at[didx_t.at[c]], ssems[p]
            ).wait()

        NEW = RPT // EC

        def ew_in_start(j, q):
            rr = pl.ds(r0 + j * EC, EC)
            pltpu.async_copy(acc_s.at[rr], msgs[2 * q], gsems[2 * q])
            pltpu.async_copy(u0_hbm.at[rr], msgs[2 * q + 1], gsems[2 * q + 1])

        def ew_in_wait(j, q):
            rr = pl.ds(r0 + j * EC, EC)
            pltpu.make_async_copy(acc_s.at[rr], msgs[2 * q], gsems[2 * q]).wait()
            pltpu.make_async_copy(
                u0_hbm.at[rr], msgs[2 * q + 1], gsems[2 * q + 1]
            ).wait()

        def ew_out_start(j, q):
            rr = pl.ds(r0 + j * EC, EC)
            pltpu.async_copy(msgs[2 * q], u_s.at[rr], ssems[2 * q])
            pltpu.async_copy(msgs[2 * q], acc_s.at[rr], ssems[2 * q + 1])

        def ew_out_wait(j, q):
            rr = pl.ds(r0 + j * EC, EC)
            pltpu.make_async_copy(msgs[2 * q], u_s.at[rr], ssems[2 * q]).wait()
            pltpu.make_async_copy(
                msgs[2 * q], acc_s.at[rr], ssems[2 * q + 1]
            ).wait()

        def step(_k, _):
            for p in range(4):
                gather_start(p, p)

            def blk(b, _c):
                for p in range(4):
                    c = 4 * b + p
                    gather_wait(c, p)
                    scat_start(c, p)

                    @pl.when(jnp.logical_and(c >= 2, c + 2 < CPT))
                    def _pref(c=c, p=p):
                        scat_wait(c - 2, (p - 2) % 4)
                        gather_start(c + 2, (p + 2) % 4)

                return _c

            lax.fori_loop(0, CPT // 4, blk, None)
            for p in range(4):
                scat_wait(CPT - 4 + p, p)
            plsc.subcore_barrier()

            ew_in_start(0, 0)
            ew_in_start(1, 1)

            for jj in range(NEW):
                q = jj % 2
                ew_in_wait(jj, q)

                def ewrow(r, _2, jj=jj, q=q):
                    bc = plsc.load_gather(
                        cexp_r,
                        [jnp.full((LANES,), jj * EC + r, jnp.int32)],
                    )
                    vals = []
                    for c0 in _COLS:
                        sl = pl.ds(c0, LANES)
                        vals.append(
                            bc * msgs[2 * q][r, sl]
                            + ALPHA * msgs[2 * q + 1][r, sl]
                        )
                    for c0, v in zip(_COLS, vals):
                        msgs[2 * q][r, pl.ds(c0, LANES)] = v
                    return _2

                lax.fori_loop(0, EC, ewrow, None, unroll=2)
                ew_out_start(jj, q)
                if jj >= 1:
                    ew_out_wait(jj - 1, 1 - q)
                    if jj + 1 < NEW:
                        ew_in_start(jj + 1, 1 - q)
            ew_out_wait(NEW - 1, (NEW - 1) % 2)
            plsc.subcore_barrier()
            return _

        lax.fori_loop(0, K, step, None)

        pltpu.sync_copy(u_s.at[rows], out_hbm.at[rows])


def _final_body(u_ref, deg_ref, o_ref):
    v = u_ref[...] * jnp.sqrt(deg_ref[...] + 1.0)
    m = jnp.max(v, axis=1, keepdims=True)
    s = jnp.log(jnp.sum(jnp.exp(v - m), axis=1, keepdims=True))
    o_ref[...] = v - m - s


def _final(uK, deg):
    rows = 1000
    return pl.pallas_call(
        _final_body,
        grid=(N // rows,),
        in_specs=[
            pl.BlockSpec((rows, D), lambda i: (i, 0)),
            pl.BlockSpec((rows, 1), lambda i: (i, 0)),
        ],
        out_specs=pl.BlockSpec((rows, NCLS), lambda i: (i, 0)),
        out_shape=jax.ShapeDtypeStruct((N, NCLS), jnp.float32),
    )(uK, deg.reshape(N_PAD, 1))


def kernel(x, edge_index, W1, b1, W2, b2):
    xp = jnp.pad(x, ((0, N_PAD - N), (0, 0)))
    pad_e = E_PAD - E
    pad_idx = N + (jnp.arange(pad_e, dtype=jnp.int32) % (N_PAD - N))
    src = jnp.concatenate([edge_index[0], pad_idx]).reshape(NS * CPT, EC)
    dst = jnp.concatenate([edge_index[1], pad_idx]).reshape(NS * CPT, EC)

    zp = _mlp(xp, W1, b1, W2, b2)
    deg = _deg_kernel(dst)
    u0 = _prep(zp, deg)
    uK = _prop_kernel(u0, deg, src, dst)
    return _final(uK, deg)


# speedup vs baseline: 45.7965x; 1.4827x over previous
import functools
import jax
import jax.numpy as jnp
from jax import lax
from jax.experimental import pallas as pl
from jax.experimental.pallas import tpu as pltpu
from jax.experimental.pallas import tpu_sc as plsc

N = 10000
E = 320000
F_IN = 128
NHID = 256
NCLS = 40
K = 10
ALPHA = 0.1

NS = 16
D = NCLS
N_PAD = 10240
RPT = N_PAD // NS
EC = 128
CPT = 160
EPT = EC * CPT
E_PAD = NS * EPT
CPT2 = CPT // 2
HALF = N_PAD // 2
HRPT = HALF // NS
EWC = 160
TCROWS = 1024
LANES = 16
_COLS = (0, 16, 24)

_mesh = plsc.VectorSubcoreMesh(core_axis_name="c", subcore_axis_name="s")
_sc_params = pltpu.CompilerParams(
    use_tc_tiling_on_sc=False, needs_layout_passes=False
)


def _mlp_body(x_ref, w1_ref, b1_ref, w2_ref, b2_ref, z_ref):
    h = jnp.maximum(
        jnp.dot(x_ref[...], w1_ref[...], preferred_element_type=jnp.float32)
        + b1_ref[...],
        0.0,
    )
    z_ref[...] = (
        jnp.dot(h, w2_ref[...], preferred_element_type=jnp.float32) + b2_ref[...]
    )


def _mlp(xp, W1, b1, W2, b2):
    return pl.pallas_call(
        _mlp_body,
        grid=(N_PAD // TCROWS,),
        in_specs=[
            pl.BlockSpec((TCROWS, F_IN), lambda i: (i, 0)),
            pl.BlockSpec((F_IN, NHID), lambda i: (0, 0)),
            pl.BlockSpec((1, NHID), lambda i: (0, 0)),
            pl.BlockSpec((NHID, NCLS), lambda i: (0, 0)),
            pl.BlockSpec((1, NCLS), lambda i: (0, 0)),
        ],
        out_specs=pl.BlockSpec((TCROWS, D), lambda i: (i, 0)),
        out_shape=jax.ShapeDtypeStruct((N_PAD, D), jnp.float32),
    )(xp, W1, b1.reshape(1, NHID), W2, b2.reshape(1, NCLS))


@functools.partial(
    pl.kernel,
    mesh=_mesh,
    compiler_params=_sc_params,
    out_type=jax.ShapeDtypeStruct((N_PAD,), jnp.float32),
    scratch_types=[
        pltpu.VMEM((4, EC), jnp.int32),
        pltpu.VMEM((EC,), jnp.float32),
        pltpu.VMEM((RPT,), jnp.float32),
        pltpu.SemaphoreType.DMA,
        pltpu.SemaphoreType.DMA,
        pltpu.SemaphoreType.DMA,
        pltpu.SemaphoreType.DMA,
        pltpu.SemaphoreType.DMA,
        pltpu.SemaphoreType.DMA,
        pltpu.SemaphoreType.DMA,
        pltpu.SemaphoreType.DMA,
        pltpu.VMEM_SHARED((N_PAD,), jnp.float32),
    ],
)
def _deg_kernel(
    dst_hbm,
    deg_hbm,
    didx_t,
    ones_t,
    zrow_t,
    l0,
    l1,
    l2,
    l3,
    s0,
    s1,
    s2,
    s3,
    deg_s,
):
    cid = lax.axis_index("c")
    sid = lax.axis_index("s")
    lsems = (l0, l1, l2, l3)
    ssems = (s0, s1, s2, s3)

    @pl.when(cid == 0)
    def _():
        for i in range(EC // LANES):
            ones_t[pl.ds(i * LANES, LANES)] = jnp.full((LANES,), 1.0, jnp.float32)
        for i in range(RPT // LANES):
            zrow_t[pl.ds(i * LANES, LANES)] = jnp.zeros((LANES,), jnp.float32)
        pltpu.sync_copy(zrow_t, deg_s.at[pl.ds(sid * RPT, RPT)])
        plsc.subcore_barrier()

        def lstart(c, p):
            pltpu.async_copy(dst_hbm.at[sid * CPT + c], didx_t.at[p], lsems[p])

        def lwait(c, p):
            pltpu.make_async_copy(
                dst_hbm.at[sid * CPT + c], didx_t.at[p], lsems[p]
            ).wait()

        def sstart(p):
            pltpu.async_copy(ones_t, deg_s.at[didx_t.at[p]], ssems[p], add=True)

        def swait(p):
            pltpu.make_async_copy(ones_t, deg_s.at[didx_t.at[p]], ssems[p]).wait()

        for p in range(4):
            lstart(p, p)

        def blk(b, _c):
            for p in range(4):
                c = 4 * b + p
                lwait(c, p)
                sstart(p)

                @pl.when(jnp.logical_and(c >= 2, c + 2 < CPT))
                def _pref(c=c, p=p):
                    swait((p - 2) % 4)
                    lstart(c + 2, (p + 2) % 4)

            return _c

        lax.fori_loop(0, CPT // 4, blk, None)
        for p in range(4):
            swait(p)
        plsc.subcore_barrier()
        pltpu.sync_copy(
            deg_s.at[pl.ds(sid * RPT, RPT)], deg_hbm.at[pl.ds(sid * RPT, RPT)]
        )


def _prep_body(z_ref, deg_ref, u0_ref):
    d = deg_ref[...] + 1.0
    u0_ref[...] = z_ref[...] * jax.lax.rsqrt(d)


def _prep(zp, deg):
    return pl.pallas_call(
        _prep_body,
        grid=(N_PAD // TCROWS,),
        in_specs=[
            pl.BlockSpec((TCROWS, D), lambda i: (i, 0)),
            pl.BlockSpec((TCROWS, 1), lambda i: (i, 0)),
        ],
        out_specs=pl.BlockSpec((TCROWS, D), lambda i: (i, 0)),
        out_shape=jax.ShapeDtypeStruct((N_PAD, D), jnp.float32),
    )(zp, deg.reshape(N_PAD, 1))


@functools.partial(
    pl.kernel,
    mesh=_mesh,
    compiler_params=_sc_params,
    out_type=(
        jax.ShapeDtypeStruct((N_PAD, D), jnp.float32),
        jax.ShapeDtypeStruct((N_PAD, D), jnp.float32),
        jax.ShapeDtypeStruct((N_PAD, D), jnp.float32),
    ),
    scratch_types=[
        pltpu.VMEM((HRPT,), jnp.float32),
        pltpu.VMEM((HRPT,), jnp.float32),
        pltpu.VMEM((CPT2, EC), jnp.int32),
        pltpu.VMEM((CPT2, EC), jnp.int32),
        pltpu.VMEM((EC, D), jnp.float32),
        pltpu.VMEM((EC, D), jnp.float32),
        pltpu.VMEM((EC, D), jnp.float32),
        pltpu.VMEM((EC, D), jnp.float32),
        pltpu.VMEM((EWC, D), jnp.float32),
        pltpu.VMEM((EWC, D), jnp.float32),
        pltpu.VMEM((EWC, D), jnp.float32),
        pltpu.VMEM((EWC, D), jnp.float32),
        pltpu.SemaphoreType.DMA,
        pltpu.SemaphoreType.DMA,
        pltpu.SemaphoreType.DMA,
        pltpu.SemaphoreType.DMA,
        pltpu.SemaphoreType.DMA,
        pltpu.SemaphoreType.DMA,
        pltpu.SemaphoreType.DMA,
        pltpu.SemaphoreType.DMA,
        pltpu.SemaphoreType.REGULAR,
        pltpu.VMEM_SHARED((N_PAD, D), jnp.float32),
        pltpu.VMEM_SHARED((N_PAD, D), jnp.float32),
    ],
)
def _prop_kernel(
    u0_hbm,
    deg_hbm,
    src_hbm,
    dst_hbm,
    out_hbm,
    accx0,
    accx1,
    cexp_r,
    degb,
    sidx_t,
    didx_t,
    msg0,
    msg1,
    msg2,
    msg3,
    aA,
    aB,
    u0b,
    zb,
    gsem0,
    gsem1,
    gsem2,
    gsem3,
    ssem0,
    ssem1,
    ssem2,
    ssem3,
    xsem,
    u_s,
    acc_s,
):
    cid = lax.axis_index("c")
    sid = lax.axis_index("s")
    msgs = (msg0, msg1, msg2, msg3)
    gsems = (gsem0, gsem1, gsem2, gsem3)
    ssems = (ssem0, ssem1, ssem2, ssem3)

    wid = cid * NS + sid

    def xbarrier():
        plsc.subcore_barrier()

        @pl.when(sid == 0)
        def _x():
            pltpu.semaphore_signal(xsem, 1, core_index=1 - cid)
            pltpu.semaphore_wait(xsem, 1)

        plsc.subcore_barrier()

    rows640 = pl.ds(sid * RPT, RPT)
    r0h = cid * HALF + sid * HRPT
    ro0 = (1 - cid) * HALF + sid * HRPT
    rowsH = pl.ds(r0h, HRPT)
    rowsO = pl.ds(ro0, HRPT)

    pltpu.sync_copy(src_hbm.at[pl.ds(wid * CPT2, CPT2)], sidx_t)
    pltpu.sync_copy(dst_hbm.at[pl.ds(wid * CPT2, CPT2)], didx_t)
    pltpu.sync_copy(deg_hbm.at[rowsH], degb)

    def cinit(i, _c):
        sl = pl.ds(i * LANES, LANES)
        cexp_r[sl] = (1.0 - ALPHA) / (degb[sl] + 1.0)
        return _c

    lax.fori_loop(0, HRPT // LANES, cinit, None)

    def zinit(r, _z):
        for c0 in _COLS:
            zb[r, pl.ds(c0, LANES)] = jnp.zeros((LANES,), jnp.float32)
        return _z

    lax.fori_loop(0, EWC, zinit, None)

    pltpu.sync_copy(u0_hbm.at[rows640], u_s.at[rows640])
    pltpu.sync_copy(u0_hbm.at[rowsH], acc_s.at[rowsH])
    for j in range(HRPT // EWC):
        pltpu.sync_copy(zb, acc_s.at[pl.ds(ro0 + j * EWC, EWC)])
    plsc.subcore_barrier()

    def gather_start(c, p):
        pltpu.async_copy(u_s.at[sidx_t.at[c]], msgs[p], gsems[p])

    def gather_wait(c, p):
        pltpu.make_async_copy(u_s.at[sidx_t.at[c]], msgs[p], gsems[p]).wait()

    def scat_start(c, p):
        pltpu.async_copy(msgs[p], acc_s.at[didx_t.at[c]], ssems[p], add=True)

    def scat_wait(c, p):
        pltpu.make_async_copy(msgs[p], acc_s.at[didx_t.at[c]], ssems[p]).wait()

    def exchange_ew(accx_out, accx_in):
        pltpu.sync_copy(acc_s.at[rowsO], accx_out.at[rowsO])
        xbarrier()
        for jj in range(HRPT // EWC):
            rr = pl.ds(r0h + jj * EWC, EWC)
            pltpu.sync_copy(acc_s.at[rr], aA)
            pltpu.sync_copy(accx_in.at[rr], aB)
            pltpu.sync_copy(u0_hbm.at[rr], u0b)

            def ewrow(r, _2, jj=jj):
                bc = plsc.load_gather(
                    cexp_r, [jnp.full((LANES,), jj * EWC + r, jnp.int32)]
                )
                vals = []
                for c0 in _COLS:
                    sl = pl.ds(c0, LANES)
                    vals.append(
                        bc * (aA[r, sl] + aB[r, sl]) + ALPHA * u0b[r, sl]
                    )
                for c0, v in zip(_COLS, vals):
                    aA[r, pl.ds(c0, LANES)] = v
                return _2

            lax.fori_loop(0, EWC, ewrow, None, unroll=2)
            pltpu.sync_copy(aA, u_s.at[rr])
            pltpu.sync_copy(aA, acc_s.at[rr])
            pltpu.sync_copy(aA, out_hbm.at[rr])
        for j in range(HRPT // EWC):
            pltpu.sync_copy(zb, acc_s.at[pl.ds(ro0 + j * EWC, EWC)])
        xbarrier()
        pltpu.sync_copy(out_hbm.at[rowsO], u_s.at[rowsO])

    def step(_k, _):
        for p in range(4):
            gather_start(p, p)

        def blk(b, _c):
            for p in range(4):
                c = 4 * b + p
                gather_wait(c, p)
                scat_start(c, p)

                @pl.when(jnp.logical_and(c >= 2, c + 2 < CPT2))
                def _pref(c=c, p=p):
                    scat_wait(c - 2, (p - 2) % 4)
                    gather_start(c + 2, (p + 2) % 4)

            return _c

        lax.fori_loop(0, CPT2 // 4, blk, None)
        for p in range(4):
            scat_wait(CPT2 - 4 + p, p)
        plsc.subcore_barrier()

        @pl.when(cid == 0)
        def _c0():
            exchange_ew(accx0, accx1)

        @pl.when(cid == 1)
        def _c1():
            exchange_ew(accx1, accx0)

        plsc.subcore_barrier()
        return _

    lax.fori_loop(0, K, step, None)


def _final_body(u_ref, deg_ref, o_ref):
    v = u_ref[...] * jnp.sqrt(deg_ref[...] + 1.0)
    m = jnp.max(v, axis=1, keepdims=True)
    s = jnp.log(jnp.sum(jnp.exp(v - m), axis=1, keepdims=True))
    o_ref[...] = v - m - s


def _final(uK, deg):
    rows = 1000
    return pl.pallas_call(
        _final_body,
        grid=(N // rows,),
        in_specs=[
            pl.BlockSpec((rows, D), lambda i: (i, 0)),
            pl.BlockSpec((rows, 1), lambda i: (i, 0)),
        ],
        out_specs=pl.BlockSpec((rows, NCLS), lambda i: (i, 0)),
        out_shape=jax.ShapeDtypeStruct((N, NCLS), jnp.float32),
    )(uK, deg.reshape(N_PAD, 1))


def kernel(x, edge_index, W1, b1, W2, b2):
    xp = jnp.pad(x, ((0, N_PAD - N), (0, 0)))
    pad_e = E_PAD - E
    pad_idx = N + (jnp.arange(pad_e, dtype=jnp.int32) % (N_PAD - N))
    src = jnp.concatenate([edge_index[0], pad_idx]).reshape(NS * CPT, EC)
    dst = jnp.concatenate([edge_index[1], pad_idx]).reshape(NS * CPT, EC)

    zp = _mlp(xp, W1, b1, W2, b2)
    deg = _deg_kernel(dst)
    u0 = _prep(zp, deg)
    uK, _unused0, _unused1 = _prop_kernel(u0, deg, src, dst)
    return _final(uK, deg)


# speedup vs baseline: 50.1058x; 1.0941x over previous
import functools
import jax
import jax.numpy as jnp
from jax import lax
from jax.experimental import pallas as pl
from jax.experimental.pallas import tpu as pltpu
from jax.experimental.pallas import tpu_sc as plsc

N = 10000
E = 320000
F_IN = 128
NHID = 256
NCLS = 40
K = 10
ALPHA = 0.1

NS = 16
D = NCLS
N_PAD = 10240
RPT = N_PAD // NS
EC = 128
CPT = 160
EPT = EC * CPT
E_PAD = NS * EPT
CPT2 = CPT // 2
HALF = N_PAD // 2
HRPT = HALF // NS
EWC = 160
TCROWS = 1024
LANES = 16
_COLS = (0, 16, 24)

_mesh = plsc.VectorSubcoreMesh(core_axis_name="c", subcore_axis_name="s")
_sc_params = pltpu.CompilerParams(
    use_tc_tiling_on_sc=False, needs_layout_passes=False
)


def _mlp_body(x_ref, w1_ref, b1_ref, w2_ref, b2_ref, z_ref):
    h = jnp.maximum(
        jnp.dot(x_ref[...], w1_ref[...], preferred_element_type=jnp.float32)
        + b1_ref[...],
        0.0,
    )
    z_ref[...] = (
        jnp.dot(h, w2_ref[...], preferred_element_type=jnp.float32) + b2_ref[...]
    )


def _mlp(xp, W1, b1, W2, b2):
    return pl.pallas_call(
        _mlp_body,
        grid=(N_PAD // TCROWS,),
        in_specs=[
            pl.BlockSpec((TCROWS, F_IN), lambda i: (i, 0)),
            pl.BlockSpec((F_IN, NHID), lambda i: (0, 0)),
            pl.BlockSpec((1, NHID), lambda i: (0, 0)),
            pl.BlockSpec((NHID, NCLS), lambda i: (0, 0)),
            pl.BlockSpec((1, NCLS), lambda i: (0, 0)),
        ],
        out_specs=pl.BlockSpec((TCROWS, D), lambda i: (i, 0)),
        out_shape=jax.ShapeDtypeStruct((N_PAD, D), jnp.float32),
    )(xp, W1, b1.reshape(1, NHID), W2, b2.reshape(1, NCLS))


@functools.partial(
    pl.kernel,
    mesh=_mesh,
    compiler_params=_sc_params,
    out_type=(
        jax.ShapeDtypeStruct((N_PAD, D), jnp.float32),
        jax.ShapeDtypeStruct((N_PAD,), jnp.float32),
        jax.ShapeDtypeStruct((N_PAD, D), jnp.float32),
        jax.ShapeDtypeStruct((N_PAD, D), jnp.float32),
        jax.ShapeDtypeStruct((N_PAD,), jnp.float32),
        jax.ShapeDtypeStruct((N_PAD,), jnp.float32),
    ),
    scratch_types=[
        pltpu.VMEM((HRPT,), jnp.float32),
        pltpu.VMEM((HRPT,), jnp.float32),
        pltpu.VMEM((HRPT,), jnp.float32),
        pltpu.VMEM((EC,), jnp.float32),
        pltpu.VMEM((CPT2, EC), jnp.int32),
        pltpu.VMEM((CPT2, EC), jnp.int32),
        pltpu.VMEM((EC, D), jnp.float32),
        pltpu.VMEM((EC, D), jnp.float32),
        pltpu.VMEM((EC, D), jnp.float32),
        pltpu.VMEM((EC, D), jnp.float32),
        pltpu.VMEM((EWC, D), jnp.float32),
        pltpu.VMEM((EWC, D), jnp.float32),
        pltpu.VMEM((HRPT, D), jnp.float32),
        pltpu.VMEM((EWC, D), jnp.float32),
        pltpu.SemaphoreType.DMA,
        pltpu.SemaphoreType.DMA,
        pltpu.SemaphoreType.DMA,
        pltpu.SemaphoreType.DMA,
        pltpu.SemaphoreType.DMA,
        pltpu.SemaphoreType.DMA,
        pltpu.SemaphoreType.DMA,
        pltpu.SemaphoreType.DMA,
        pltpu.SemaphoreType.REGULAR,
        pltpu.VMEM_SHARED((N_PAD, D), jnp.float32),
        pltpu.VMEM_SHARED((N_PAD, D), jnp.float32),
        pltpu.VMEM_SHARED((N_PAD,), jnp.float32),
    ],
)
def _prop_kernel(
    z_hbm,
    src_hbm,
    dst_hbm,
    out_hbm,
    deg_hbm,
    accx0,
    accx1,
    degx0,
    degx1,
    cexp_r,
    degb,
    degc,
    ones_t,
    sidx_t,
    didx_t,
    msg0,
    msg1,
    msg2,
    msg3,
    aA,
    aB,
    g_t,
    zb,
    gsem0,
    gsem1,
    gsem2,
    gsem3,
    ssem0,
    ssem1,
    ssem2,
    ssem3,
    xsem,
    u_s,
    acc_s,
    deg_s,
):
    cid = lax.axis_index("c")
    sid = lax.axis_index("s")
    msgs = (msg0, msg1, msg2, msg3)
    gsems = (gsem0, gsem1, gsem2, gsem3)
    ssems = (ssem0, ssem1, ssem2, ssem3)

    wid = cid * NS + sid

    def xbarrier():
        plsc.subcore_barrier()

        @pl.when(sid == 0)
        def _x():
            pltpu.semaphore_signal(xsem, 1, core_index=1 - cid)
            pltpu.semaphore_wait(xsem, 1)

        plsc.subcore_barrier()

    rows640 = pl.ds(sid * RPT, RPT)
    r0h = cid * HALF + sid * HRPT
    ro0 = (1 - cid) * HALF + sid * HRPT
    rowsH = pl.ds(r0h, HRPT)
    rowsO = pl.ds(ro0, HRPT)

    pltpu.sync_copy(src_hbm.at[pl.ds(wid * CPT2, CPT2)], sidx_t)
    pltpu.sync_copy(dst_hbm.at[pl.ds(wid * CPT2, CPT2)], didx_t)

    for i in range(EC // LANES):
        ones_t[pl.ds(i * LANES, LANES)] = jnp.full((LANES,), 1.0, jnp.float32)

    def zinit(r, _z):
        for c0 in _COLS:
            zb[r, pl.ds(c0, LANES)] = jnp.zeros((LANES,), jnp.float32)
        return _z

    lax.fori_loop(0, EWC, zinit, None)

    def dzero(i, _c):
        degb[pl.ds(i * LANES, LANES)] = jnp.zeros((LANES,), jnp.float32)
        return _c

    lax.fori_loop(0, HRPT // LANES, dzero, None)
    pltpu.sync_copy(degb, deg_s.at[pl.ds(sid * RPT, HRPT)])
    pltpu.sync_copy(degb, deg_s.at[pl.ds(sid * RPT + HRPT, HRPT)])
    plsc.subcore_barrier()

    ssems_t = (ssem0, ssem1, ssem2, ssem3)

    def hstart(c, p):
        pltpu.async_copy(ones_t, deg_s.at[didx_t.at[c]], ssems_t[p], add=True)

    def hwait(c, p):
        pltpu.make_async_copy(
            ones_t, deg_s.at[didx_t.at[c]], ssems_t[p]
        ).wait()

    def hblk(b, _c):
        for p in range(4):
            c = 4 * b + p

            @pl.when(c >= 4)
            def _w(c=c, p=p):
                hwait(c - 4, p)

            hstart(c, p)
        return _c

    lax.fori_loop(0, CPT2 // 4, hblk, None)
    for p in range(4):
        hwait(CPT2 - 4 + p, p)
    plsc.subcore_barrier()

    def _rsqrt16(x):
        i = plsc.bitcast(x, jnp.int32)
        i = 0x5F3759DF - (i >> 1)
        y = plsc.bitcast(i, jnp.float32)
        for _ in range(3):
            y = y * (1.5 - 0.5 * x * y * y)
        return y

    def init_exchange(degx_out, degx_in):
        pltpu.sync_copy(deg_s.at[rowsO], degx_out.at[rowsO])
        xbarrier()
        pltpu.sync_copy(deg_s.at[rowsH], degb)
        pltpu.sync_copy(degx_in.at[rowsH], degc)

        def dinit(i, _c):
            sl = pl.ds(i * LANES, LANES)
            dtot = degb[sl] + degc[sl] + 1.0
            cexp_r[sl] = (1.0 - ALPHA) / dtot
            degb[sl] = dtot
            degc[sl] = _rsqrt16(dtot)
            return _c

        lax.fori_loop(0, HRPT // LANES, dinit, None)
        pltpu.sync_copy(degb, deg_hbm.at[rowsH])

        for jj in range(HRPT // EWC):
            rr = pl.ds(r0h + jj * EWC, EWC)
            pltpu.sync_copy(z_hbm.at[rr], aA)

            def u0row(r, _2, jj=jj):
                bc = plsc.load_gather(
                    degc, [jnp.full((LANES,), jj * EWC + r, jnp.int32)]
                )
                vals = []
                for c0 in _COLS:
                    vals.append(bc * aA[r, pl.ds(c0, LANES)])
                for c0, v in zip(_COLS, vals):
                    aA[r, pl.ds(c0, LANES)] = v
                    g_t[jj * EWC + r, pl.ds(c0, LANES)] = ALPHA * v
                return _2

            lax.fori_loop(0, EWC, u0row, None)
            pltpu.sync_copy(aA, u_s.at[rr])
            pltpu.sync_copy(aA, acc_s.at[rr])
            pltpu.sync_copy(aA, out_hbm.at[rr])
        for j in range(HRPT // EWC):
            pltpu.sync_copy(zb, acc_s.at[pl.ds(ro0 + j * EWC, EWC)])
        xbarrier()
        pltpu.sync_copy(out_hbm.at[rowsO], u_s.at[rowsO])

    @pl.when(cid == 0)
    def _i0():
        init_exchange(degx0, degx1)

    @pl.when(cid == 1)
    def _i1():
        init_exchange(degx1, degx0)

    plsc.subcore_barrier()

    def gather_start(c, p):
        pltpu.async_copy(u_s.at[sidx_t.at[c]], msgs[p], gsems[p])

    def gather_wait(c, p):
        pltpu.make_async_copy(u_s.at[sidx_t.at[c]], msgs[p], gsems[p]).wait()

    def scat_start(c, p):
        pltpu.async_copy(msgs[p], acc_s.at[didx_t.at[c]], ssems[p], add=True)

    def scat_wait(c, p):
        pltpu.make_async_copy(msgs[p], acc_s.at[didx_t.at[c]], ssems[p]).wait()

    def exchange_ew(accx_out, accx_in):
        pltpu.sync_copy(acc_s.at[rowsO], accx_out.at[rowsO])
        xbarrier()
        for jj in range(HRPT // EWC):
            rr = pl.ds(r0h + jj * EWC, EWC)
            pltpu.sync_copy(acc_s.at[rr], aA)
            pltpu.sync_copy(accx_in.at[rr], aB)

            def ewrow(r, _2, jj=jj):
                bc = plsc.load_gather(
                    cexp_r, [jnp.full((LANES,), jj * EWC + r, jnp.int32)]
                )
                vals = []
                for c0 in _COLS:
                    sl = pl.ds(c0, LANES)
                    vals.append(
                        bc * (aA[r, sl] + aB[r, sl])
                        + g_t[jj * EWC + r, sl]
                    )
                for c0, v in zip(_COLS, vals):
                    aA[r, pl.ds(c0, LANES)] = v
                return _2

            lax.fori_loop(0, EWC, ewrow, None, unroll=2)
            pltpu.sync_copy(aA, u_s.at[rr])
            pltpu.sync_copy(aA, acc_s.at[rr])
            pltpu.sync_copy(aA, out_hbm.at[rr])
        for j in range(HRPT // EWC):
            pltpu.sync_copy(zb, acc_s.at[pl.ds(ro0 + j * EWC, EWC)])
        xbarrier()
        pltpu.sync_copy(out_hbm.at[rowsO], u_s.at[rowsO])

    def step(_k, _):
        for p in range(4):
            gather_start(p, p)

        def blk(b, _c):
            for p in range(4):
                c = 4 * b + p
                gather_wait(c, p)
                scat_start(c, p)

                @pl.when(jnp.logical_and(c >= 2, c + 2 < CPT2))
                def _pref(c=c, p=p):
                    scat_wait(c - 2, (p - 2) % 4)
                    gather_start(c + 2, (p + 2) % 4)

            return _c

        lax.fori_loop(0, CPT2 // 4, blk, None)
        for p in range(4):
            scat_wait(CPT2 - 4 + p, p)
        plsc.subcore_barrier()

        @pl.when(cid == 0)
        def _c0():
            exchange_ew(accx0, accx1)

        @pl.when(cid == 1)
        def _c1():
            exchange_ew(accx1, accx0)

        plsc.subcore_barrier()
        return _

    lax.fori_loop(0, K, step, None)


def _final_body(u_ref, deg_ref, o_ref):
    v = u_ref[...] * jnp.sqrt(deg_ref[...])
    m = jnp.max(v, axis=1, keepdims=True)
    s = jnp.log(jnp.sum(jnp.exp(v - m), axis=1, keepdims=True))
    o_ref[...] = v - m - s


def _final(uK, deg):
    rows = 1000
    return pl.pallas_call(
        _final_body,
        grid=(N // rows,),
        in_specs=[
            pl.BlockSpec((rows, D), lambda i: (i, 0)),
            pl.BlockSpec((rows, 1), lambda i: (i, 0)),
        ],
        out_specs=pl.BlockSpec((rows, NCLS), lambda i: (i, 0)),
        out_shape=jax.ShapeDtypeStruct((N, NCLS), jnp.float32),
    )(uK, deg.reshape(N_PAD, 1))


def kernel(x, edge_index, W1, b1, W2, b2):
    xp = jnp.pad(x, ((0, N_PAD - N), (0, 0)))
    pad_e = E_PAD - E
    pad_idx = N + (jnp.arange(pad_e, dtype=jnp.int32) % (N_PAD - N))
    src = jnp.concatenate([edge_index[0], pad_idx]).reshape(NS * CPT, EC)
    dst = jnp.concatenate([edge_index[1], pad_idx]).reshape(NS * CPT, EC)

    zp = _mlp(xp, W1, b1, W2, b2)
    uK, deg, _x0, _x1, _x2, _x3 = _prop_kernel(zp, src, dst)
    return _final(uK, deg)
